# baseline (speedup 1.0000x reference)
import jax
import jax.numpy as jnp
from jax.experimental import pallas as pl
from jax.experimental.pallas import tpu as pltpu

N = 4096
H = 128
T = 50
BM = 512


def _body(aidx_ref, treat_ref, z_ref, w_ref, b_ref, adj_t_ref, adj_b_ref,
          out_ref, xw_ref):
    @pl.when(pl.program_id(0) == 0)
    def _compute_xw():
        row_ids = jax.lax.broadcasted_iota(jnp.int32, (T, 1), 0)
        sel = (row_ids == aidx_ref[0]).astype(jnp.float32)
        m = sel * w_ref[H:H + 1, :]
        zw = jnp.dot(z_ref[...], w_ref[:H, :], preferred_element_type=jnp.float32)
        xw_ref[...] = zw + jnp.dot(treat_ref[...], m,
                                   preferred_element_type=jnp.float32)

    acc_t = jnp.dot(adj_t_ref[...], xw_ref[...], preferred_element_type=jnp.float32)
    out_ref[:BM // 2, :] = jnp.maximum(acc_t + b_ref[...], 0.0)
    acc_b = jnp.dot(adj_b_ref[...], xw_ref[...], preferred_element_type=jnp.float32)
    out_ref[BM // 2:, :] = jnp.maximum(acc_b + b_ref[...], 0.0)


@jax.jit
def kernel(t, z, treatments, adj, W, b):
    a_idx = jnp.clip((t * (T - 1)).astype(jnp.int32), 0, T - 1)
    treat2d = treatments[:, :, 0]
    b2d = b.reshape(1, H)

    grid = (N // BM,)
    out = pl.pallas_call(
        _body,
        grid_spec=pltpu.PrefetchScalarGridSpec(
            num_scalar_prefetch=1,
            grid=grid,
            in_specs=[
                pl.BlockSpec((N, T), lambda i, s: (0, 0)),
                pl.BlockSpec((N, H), lambda i, s: (0, 0)),
                pl.BlockSpec((H + 1, H), lambda i, s: (0, 0)),
                pl.BlockSpec((1, H), lambda i, s: (0, 0)),
                pl.BlockSpec((BM // 2, N), lambda i, s: (2 * i, 0)),
                pl.BlockSpec((BM // 2, N), lambda i, s: (2 * i + 1, 0)),
            ],
            out_specs=pl.BlockSpec((BM, H), lambda i, s: (i, 0)),
            scratch_shapes=[pltpu.VMEM((N, H), jnp.float32)],
        ),
        out_shape=jax.ShapeDtypeStruct((N, H), jnp.float32),
        compiler_params=pltpu.CompilerParams(
            dimension_semantics=("arbitrary",),
        ),
    )(a_idx.reshape(1), treat2d, z, W, b2d, adj, adj)
    return out


# speedup vs baseline: 1.0390x; 1.0390x over previous
import jax
import jax.numpy as jnp
from jax.experimental import pallas as pl
from jax.experimental.pallas import tpu as pltpu

N = 4096
H = 128
T = 50
BM = 512


def _body(aidx_ref, treat_ref, z_ref, w_ref, b_ref, adj_ref, out_ref, xw_ref):
    @pl.when(pl.program_id(0) == 0)
    def _compute_xw():
        row_ids = jax.lax.broadcasted_iota(jnp.int32, (T, 1), 0)
        sel = (row_ids == aidx_ref[0]).astype(jnp.float32)
        m = sel * w_ref[H:H + 1, :]
        zw = jnp.dot(z_ref[...], w_ref[:H, :], preferred_element_type=jnp.float32)
        xw_ref[...] = zw + jnp.dot(treat_ref[...], m,
                                   preferred_element_type=jnp.float32)

    acc = jnp.dot(adj_ref[...], xw_ref[...], preferred_element_type=jnp.float32)
    out_ref[...] = jnp.maximum(acc + b_ref[...], 0.0)


@jax.jit
def kernel(t, z, treatments, adj, W, b):
    a_idx = jnp.clip((t * (T - 1)).astype(jnp.int32), 0, T - 1)
    treat2d = treatments[:, :, 0]
    b2d = b.reshape(1, H)

    grid = (N // BM,)
    out = pl.pallas_call(
        _body,
        grid_spec=pltpu.PrefetchScalarGridSpec(
            num_scalar_prefetch=1,
            grid=grid,
            in_specs=[
                pl.BlockSpec((N, T), lambda i, s: (0, 0)),
                pl.BlockSpec((N, H), lambda i, s: (0, 0)),
                pl.BlockSpec((H + 1, H), lambda i, s: (0, 0)),
                pl.BlockSpec((1, H), lambda i, s: (0, 0)),
                pl.BlockSpec((BM, N), lambda i, s: (i, 0)),
            ],
            out_specs=pl.BlockSpec((BM, H), lambda i, s: (i, 0)),
            scratch_shapes=[pltpu.VMEM((N, H), jnp.float32)],
        ),
        out_shape=jax.ShapeDtypeStruct((N, H), jnp.float32),
        compiler_params=pltpu.CompilerParams(
            dimension_semantics=("arbitrary",),
        ),
    )(a_idx.reshape(1), treat2d, z, W, b2d, adj)
    return out


# speedup vs baseline: 1.0789x; 1.0384x over previous
import jax
import jax.numpy as jnp
from jax.experimental import pallas as pl
from jax.experimental.pallas import tpu as pltpu

N = 4096
H = 128
T = 50
BM = 512


def _body(at_ref, z_ref, w_ref, b_ref, adj_ref, out_ref, xw_ref):
    @pl.when(pl.program_id(0) == 0)
    def _compute_xw():
        zw = jnp.dot(z_ref[...], w_ref[:H, :], preferred_element_type=jnp.float32)
        xw_ref[...] = zw + at_ref[...] * w_ref[H:H + 1, :]

    acc = jnp.dot(adj_ref[...], xw_ref[...], preferred_element_type=jnp.float32)
    out_ref[...] = jnp.maximum(acc + b_ref[...], 0.0)


@jax.jit
def kernel(t, z, treatments, adj, W, b):
    a_idx = jnp.clip((t * (T - 1)).astype(jnp.int32), 0, T - 1)
    a_t = jnp.take(treatments, a_idx, axis=1)
    b2d = b.reshape(1, H)

    grid = (N // BM,)
    out = pl.pallas_call(
        _body,
        grid=grid,
        in_specs=[
            pl.BlockSpec((N, 1), lambda i: (0, 0)),
            pl.BlockSpec((N, H), lambda i: (0, 0)),
            pl.BlockSpec((H + 1, H), lambda i: (0, 0)),
            pl.BlockSpec((1, H), lambda i: (0, 0)),
            pl.BlockSpec((BM, N), lambda i: (i, 0)),
        ],
        out_specs=pl.BlockSpec((BM, H), lambda i: (i, 0)),
        scratch_shapes=[pltpu.VMEM((N, H), jnp.float32)],
        out_shape=jax.ShapeDtypeStruct((N, H), jnp.float32),
        compiler_params=pltpu.CompilerParams(
            dimension_semantics=("arbitrary",),
        ),
    )(a_t, z, W, b2d, adj)
    return out
